# Initial kernel scaffold; baseline (speedup 1.0000x reference)
import jax
import jax.numpy as jnp
from jax.experimental import pallas as pl


def kernel(x, edge_index, batch, params):
    raise NotImplementedError("write your pallas kernel here")


# speedup vs baseline: 4.7220x; 4.7220x over previous
import functools

import jax
import jax.numpy as jnp
from jax import lax
from jax.experimental import pallas as pl
from jax.experimental.pallas import tpu as pltpu
from jax.experimental.pallas import tpu_sc as plsc

N = 10000
E = 320000
D = 128
H = 128
ID_DIM = 16
G = 64

NC = 2
NS = 16
NW = NC * NS

K = 128
CH = (E + NW * K - 1) // (NW * K)
PT = CH * K
E_PAD = NW * PT
N_SP = 10240
ROWZ = N_SP // NS

_f32 = jnp.float32


def _sc_mesh():
    return plsc.VectorSubcoreMesh(
        core_axis_name="c", subcore_axis_name="s", num_cores=NC, num_subcores=NS
    )


@functools.partial(
    pl.kernel,
    out_type=jax.ShapeDtypeStruct((NC, N_SP, H), _f32),
    mesh=_sc_mesh(),
    scratch_types=[
        pltpu.VMEM((CH, K), jnp.int32),
        pltpu.VMEM((CH, K), jnp.int32),
        pltpu.VMEM((K, H), _f32),
        pltpu.VMEM_SHARED((N_SP, H), _f32),
        pltpu.SemaphoreType.DMA,
    ],
)
def _sc_agg(z_hbm, src_hbm, dst_hbm, zeros_hbm, out_hbm,
            src_v, dst_v, rows_v, agg_sh, sem):
    cid = lax.axis_index("c")
    sid = lax.axis_index("s")
    wid = cid * NS + sid
    pltpu.sync_copy(zeros_hbm, agg_sh.at[pl.ds(sid * ROWZ, ROWZ)])
    pltpu.sync_copy(src_hbm.at[wid], src_v)
    pltpu.sync_copy(dst_hbm.at[wid], dst_v)
    plsc.subcore_barrier()

    def body(c, carry):
        pltpu.async_copy(z_hbm.at[src_v.at[c]], rows_v, sem).wait()
        pltpu.sync_copy(rows_v, agg_sh.at[dst_v.at[c]], add=True)
        return carry

    lax.fori_loop(0, CH, body, 0)
    plsc.subcore_barrier()
    pltpu.sync_copy(agg_sh.at[pl.ds(sid * ROWZ, ROWZ)],
                    out_hbm.at[cid, pl.ds(sid * ROWZ, ROWZ)])


def _mlp_bn(t, w2, b2, gamma, beta):
    h2 = jnp.dot(t, w2, preferred_element_type=_f32) + b2
    mean = jnp.mean(h2, axis=0, keepdims=True)
    cen = h2 - mean
    var = jnp.mean(cen * cen, axis=0, keepdims=True)
    return jnp.maximum(cen * lax.rsqrt(var + 1e-5) * gamma + beta, 0.0)


def _layer0_body(x_ref, p_ref, w1_ref, b1_ref, w2_ref, b2_ref, g_ref, be_ref,
                 h_ref, u16_ref):
    pre = x_ref[...] + p_ref[0, :N] + p_ref[1, :N]
    t = jnp.maximum(jnp.dot(pre, w1_ref[...], preferred_element_type=_f32)
                    + b1_ref[...], 0.0)
    h_ref[...] = _mlp_bn(t, w2_ref[...], b2_ref[...], g_ref[...], be_ref[...])
    u16_ref[...] = pre[:, :ID_DIM]


_tc_layer0 = pl.pallas_call(
    _layer0_body,
    out_shape=(jax.ShapeDtypeStruct((N, H), _f32),
               jax.ShapeDtypeStruct((N, ID_DIM), _f32)))


def _layer_body(u16_ref, h_ref, p_ref, w1a_ref, w1b_ref, b1_ref,
                w2_ref, b2_ref, g_ref, be_ref, o_ref):
    pre = h_ref[...] + p_ref[0, :N] + p_ref[1, :N]
    t = jnp.maximum(
        jnp.dot(u16_ref[...], w1a_ref[...], preferred_element_type=_f32)
        + jnp.dot(pre, w1b_ref[...], preferred_element_type=_f32)
        + b1_ref[...], 0.0)
    o_ref[...] = _mlp_bn(t, w2_ref[...], b2_ref[...], g_ref[...], be_ref[...])


_tc_layer = pl.pallas_call(
    _layer_body, out_shape=jax.ShapeDtypeStruct((N, H), _f32))


def _pool_body(z_ref, b_ref, o_ref):
    bid = b_ref[...]
    gid = lax.broadcasted_iota(jnp.int32, (G, N), 0)
    mask = (gid == bid).astype(_f32)
    o_ref[...] = jnp.dot(mask, z_ref[...], preferred_element_type=_f32,
                         precision=lax.Precision.HIGHEST)


_tc_pool = pl.pallas_call(
    _pool_body, out_shape=jax.ShapeDtypeStruct((G, 3 * H), _f32))


def kernel(x, edge_index, batch, params):
    src = edge_index[0]
    dst = edge_index[1]
    src_p = jnp.concatenate(
        [src, jnp.zeros((E_PAD - E,), jnp.int32)]).reshape(NW, CH, K)
    dst_p = jnp.concatenate(
        [dst, jnp.full((E_PAD - E,), N, jnp.int32)]).reshape(NW, CH, K)
    zeros = jnp.zeros((ROWZ, H), _f32)

    def vec(v):
        return v.reshape(1, H)

    p = _sc_agg(x, src_p, dst_p, zeros)
    pm = params[0]
    h, u16 = _tc_layer0(x, p, pm["W1"], vec(pm["b1"]), pm["W2"], vec(pm["b2"]),
                        vec(pm["gamma"]), vec(pm["beta"]))
    hs = [h]
    for l in (1, 2):
        p = _sc_agg(h, src_p, dst_p, zeros)
        pm = params[l]
        h = _tc_layer(u16, h, p, pm["W1"][:ID_DIM], pm["W1"][ID_DIM:],
                      vec(pm["b1"]), pm["W2"], vec(pm["b2"]),
                      vec(pm["gamma"]), vec(pm["beta"]))
        hs.append(h)

    z_out = jnp.concatenate(hs, axis=1)
    g_out = _tc_pool(z_out, batch.reshape(1, N))
    return (z_out, g_out)


# speedup vs baseline: 10.5693x; 2.2383x over previous
import functools

import jax
import jax.numpy as jnp
from jax import lax
from jax.experimental import pallas as pl
from jax.experimental.pallas import tpu as pltpu
from jax.experimental.pallas import tpu_sc as plsc

N = 10000
E = 320000
D = 128
H = 128
ID_DIM = 16
G = 64

NC = 2
NS = 16
NW = NC * NS

K = 128
NBUF = 2
RG = 4
SGC = RG * NBUF
SG = -(-E // (NW * K * SGC))
CH = SG * SGC
PT = CH * K
E_PAD = NW * PT
N_SP = 10240
ROWZ = N_SP // NS

_f32 = jnp.float32


def _sc_mesh():
    return plsc.VectorSubcoreMesh(
        core_axis_name="c", subcore_axis_name="s", num_cores=NC, num_subcores=NS
    )


@functools.partial(
    pl.kernel,
    out_type=jax.ShapeDtypeStruct((NC, N_SP, H), _f32),
    mesh=_sc_mesh(),
    scratch_types=[
        pltpu.VMEM((SGC, K), jnp.int32),
        pltpu.VMEM((SGC, K), jnp.int32),
        pltpu.VMEM((K, H), _f32),
        pltpu.VMEM((K, H), _f32),
        pltpu.VMEM_SHARED((N_SP, H), _f32),
        pltpu.SemaphoreType.DMA,
        pltpu.SemaphoreType.DMA,
    ],
)
def _sc_agg(z_hbm, src_hbm, dst_hbm, zeros_hbm, out_hbm,
            src_v, dst_v, rv0, rv1, agg_sh, s0, s1):
    rows_v = (rv0, rv1)
    sems = (s0, s1)
    cid = lax.axis_index("c")
    sid = lax.axis_index("s")
    wid = cid * NS + sid
    pltpu.sync_copy(zeros_hbm, agg_sh.at[pl.ds(sid * ROWZ, ROWZ)])
    plsc.subcore_barrier()

    def body(sg, carry):
        pltpu.sync_copy(src_hbm.at[wid, sg], src_v)
        pltpu.sync_copy(dst_hbm.at[wid, sg], dst_v)
        descs = [
            pltpu.async_copy(z_hbm.at[src_v.at[b]], rows_v[b], sems[b])
            for b in range(NBUF)
        ]
        for c in range(SGC):
            b = c % NBUF
            descs[b].wait()
            pltpu.sync_copy(rows_v[b], agg_sh.at[dst_v.at[c]], add=True)
            if c + NBUF < SGC:
                descs[b] = pltpu.async_copy(
                    z_hbm.at[src_v.at[c + NBUF]], rows_v[b], sems[b])
        return carry

    lax.fori_loop(0, SG, body, 0)
    plsc.subcore_barrier()
    pltpu.sync_copy(agg_sh.at[pl.ds(sid * ROWZ, ROWZ)],
                    out_hbm.at[cid, pl.ds(sid * ROWZ, ROWZ)])


def _mlp_bn(t, w2, b2, gamma, beta):
    h2 = jnp.dot(t, w2, preferred_element_type=_f32) + b2
    mean = jnp.mean(h2, axis=0, keepdims=True)
    cen = h2 - mean
    var = jnp.mean(cen * cen, axis=0, keepdims=True)
    return jnp.maximum(cen * lax.rsqrt(var + 1e-5) * gamma + beta, 0.0)


def _layer0_body(x_ref, p_ref, w1_ref, b1_ref, w2_ref, b2_ref, g_ref, be_ref,
                 h_ref, u16_ref):
    pre = x_ref[...] + p_ref[0, :N] + p_ref[1, :N]
    t = jnp.maximum(jnp.dot(pre, w1_ref[...], preferred_element_type=_f32)
                    + b1_ref[...], 0.0)
    h_ref[...] = _mlp_bn(t, w2_ref[...], b2_ref[...], g_ref[...], be_ref[...])
    u16_ref[...] = pre[:, :ID_DIM]


_tc_layer0 = pl.pallas_call(
    _layer0_body,
    out_shape=(jax.ShapeDtypeStruct((N, H), _f32),
               jax.ShapeDtypeStruct((N, ID_DIM), _f32)))


def _layer_body(u16_ref, h_ref, p_ref, w1a_ref, w1b_ref, b1_ref,
                w2_ref, b2_ref, g_ref, be_ref, o_ref):
    pre = h_ref[...] + p_ref[0, :N] + p_ref[1, :N]
    t = jnp.maximum(
        jnp.dot(u16_ref[...], w1a_ref[...], preferred_element_type=_f32)
        + jnp.dot(pre, w1b_ref[...], preferred_element_type=_f32)
        + b1_ref[...], 0.0)
    o_ref[...] = _mlp_bn(t, w2_ref[...], b2_ref[...], g_ref[...], be_ref[...])


_tc_layer = pl.pallas_call(
    _layer_body, out_shape=jax.ShapeDtypeStruct((N, H), _f32))


def _pool_body(z_ref, b_ref, o_ref):
    bid = b_ref[...]
    gid = lax.broadcasted_iota(jnp.int32, (G, N), 0)
    mask = (gid == bid).astype(_f32)
    o_ref[...] = jnp.dot(mask, z_ref[...], preferred_element_type=_f32,
                         precision=lax.Precision.HIGHEST)


_tc_pool = pl.pallas_call(
    _pool_body, out_shape=jax.ShapeDtypeStruct((G, 3 * H), _f32))


def kernel(x, edge_index, batch, params):
    src = edge_index[0]
    dst = edge_index[1]
    pad = jnp.arange(E_PAD - E, dtype=jnp.int32)
    src_p = jnp.concatenate([src, pad % N]).reshape(NW, SG, SGC, K)
    dst_p = jnp.concatenate([dst, pad % (N_SP - N) + N]).reshape(NW, SG, SGC, K)
    zeros = jnp.zeros((ROWZ, H), _f32)

    def vec(v):
        return v.reshape(1, H)

    p = _sc_agg(x, src_p, dst_p, zeros)
    pm = params[0]
    h, u16 = _tc_layer0(x, p, pm["W1"], vec(pm["b1"]), pm["W2"], vec(pm["b2"]),
                        vec(pm["gamma"]), vec(pm["beta"]))
    hs = [h]
    for l in (1, 2):
        p = _sc_agg(h, src_p, dst_p, zeros)
        pm = params[l]
        h = _tc_layer(u16, h, p, pm["W1"][:ID_DIM], pm["W1"][ID_DIM:],
                      vec(pm["b1"]), pm["W2"], vec(pm["b2"]),
                      vec(pm["gamma"]), vec(pm["beta"]))
        hs.append(h)

    z_out = jnp.concatenate(hs, axis=1)
    g_out = _tc_pool(z_out, batch.reshape(1, N))
    return (z_out, g_out)


# speedup vs baseline: 12.2781x; 1.1617x over previous
import functools

import jax
import jax.numpy as jnp
from jax import lax
from jax.experimental import pallas as pl
from jax.experimental.pallas import tpu as pltpu
from jax.experimental.pallas import tpu_sc as plsc

N = 10000
E = 320000
D = 128
H = 128
ID_DIM = 16
G = 64

NC = 2
NS = 16
NW = NC * NS

K = 128
NBUF = 2
SGC = 8
SG = -(-E // (NW * K * SGC))
CH = SG * SGC
PT = CH * K
E_PAD = NW * PT
N_SP = 10112
ROWZ = N_SP // NS

_f32 = jnp.float32


def _sc_mesh():
    return plsc.VectorSubcoreMesh(
        core_axis_name="c", subcore_axis_name="s", num_cores=NC, num_subcores=NS
    )


@functools.partial(
    pl.kernel,
    out_type=jax.ShapeDtypeStruct((NC, N_SP, H), _f32),
    mesh=_sc_mesh(),
    scratch_types=[
        pltpu.VMEM((2, SGC, K), jnp.int32),
        pltpu.VMEM((2, SGC, K), jnp.int32),
        pltpu.VMEM((K, H), _f32),
        pltpu.VMEM((K, H), _f32),
        pltpu.VMEM_SHARED((N_SP, H), _f32),
        pltpu.SemaphoreType.DMA,
        pltpu.SemaphoreType.DMA,
        pltpu.SemaphoreType.DMA,
        pltpu.SemaphoreType.DMA,
    ],
)
def _sc_agg(z_hbm, sd_hbm, zeros_hbm, out_hbm,
            idx0, idx1, rv0, rv1, agg_sh, s0, s1, i0, i1):
    rows_v = (rv0, rv1)
    sems = (s0, s1)
    idx = (idx0, idx1)
    isems = (i0, i1)
    cid = lax.axis_index("c")
    sid = lax.axis_index("s")
    wid = cid * NS + sid
    pltpu.sync_copy(zeros_hbm, agg_sh.at[pl.ds(sid * ROWZ, ROWZ)])
    plsc.subcore_barrier()

    pltpu.async_copy(sd_hbm.at[wid, 0], idx0, i0).wait()
    for b in range(NBUF):
        pltpu.async_copy(z_hbm.at[idx0.at[0, b]], rows_v[b], sems[b])
    pltpu.async_copy(sd_hbm.at[wid, 1], idx1, i1)

    def window(i, carry):
        nxt = jnp.minimum(2 * i + 2, SG - 1)
        for half in range(2):
            cur = idx[half]
            oth = idx[1 - half]
            for c in range(SGC):
                b = c % NBUF
                pltpu.make_async_copy(
                    z_hbm.at[pl.ds(0, K)], rows_v[b], sems[b]).wait()
                pltpu.sync_copy(rows_v[b], agg_sh.at[cur.at[1, c]], add=True)
                if c + NBUF < SGC:
                    pltpu.async_copy(
                        z_hbm.at[cur.at[0, c + NBUF]], rows_v[b], sems[b])
                else:
                    cn = c + NBUF - SGC
                    if half == 0:
                        if cn == 0:
                            pltpu.make_async_copy(
                                sd_hbm.at[wid, 0], oth, isems[1]).wait()
                        pltpu.async_copy(
                            z_hbm.at[oth.at[0, cn]], rows_v[b], sems[b])
                    else:

                        @pl.when(i < SG // 2 - 1)
                        def _():
                            if cn == 0:
                                pltpu.make_async_copy(
                                    sd_hbm.at[wid, 0], oth, isems[0]).wait()
                            pltpu.async_copy(
                                z_hbm.at[oth.at[0, cn]], rows_v[b], sems[b])
            if half == 0:
                @pl.when(i < SG // 2 - 1)
                def _():
                    pltpu.async_copy(sd_hbm.at[wid, nxt], idx0, i0)

        @pl.when(i < SG // 2 - 1)
        def _():
            pltpu.async_copy(sd_hbm.at[wid, nxt + 1], idx1, i1)

        return carry

    lax.fori_loop(0, SG // 2, window, 0)
    plsc.subcore_barrier()
    pltpu.sync_copy(agg_sh.at[pl.ds(sid * ROWZ, ROWZ)],
                    out_hbm.at[cid, pl.ds(sid * ROWZ, ROWZ)])


def _mlp_bn(t, w2, b2, gamma, beta):
    h2 = jnp.dot(t, w2, preferred_element_type=_f32) + b2
    mean = jnp.mean(h2, axis=0, keepdims=True)
    cen = h2 - mean
    var = jnp.mean(cen * cen, axis=0, keepdims=True)
    return jnp.maximum(cen * lax.rsqrt(var + 1e-5) * gamma + beta, 0.0)


def _layer0_body(x_ref, p_ref, w1_ref, b1_ref, w2_ref, b2_ref, g_ref, be_ref,
                 h_ref, u16_ref):
    pre = x_ref[...] + p_ref[0, :N] + p_ref[1, :N]
    t = jnp.maximum(jnp.dot(pre, w1_ref[...], preferred_element_type=_f32)
                    + b1_ref[...], 0.0)
    h_ref[...] = _mlp_bn(t, w2_ref[...], b2_ref[...], g_ref[...], be_ref[...])
    u16_ref[...] = pre[:, :ID_DIM]


_tc_layer0 = pl.pallas_call(
    _layer0_body,
    out_shape=(jax.ShapeDtypeStruct((N, H), _f32),
               jax.ShapeDtypeStruct((N, ID_DIM), _f32)))


def _layer_body(u16_ref, h_ref, p_ref, w1a_ref, w1b_ref, b1_ref,
                w2_ref, b2_ref, g_ref, be_ref, o_ref):
    pre = h_ref[...] + p_ref[0, :N] + p_ref[1, :N]
    t = jnp.maximum(
        jnp.dot(u16_ref[...], w1a_ref[...], preferred_element_type=_f32)
        + jnp.dot(pre, w1b_ref[...], preferred_element_type=_f32)
        + b1_ref[...], 0.0)
    o_ref[...] = _mlp_bn(t, w2_ref[...], b2_ref[...], g_ref[...], be_ref[...])


_tc_layer = pl.pallas_call(
    _layer_body, out_shape=jax.ShapeDtypeStruct((N, H), _f32))


def _pool_body(z_ref, b_ref, o_ref):
    bid = b_ref[...]
    gid = lax.broadcasted_iota(jnp.int32, (G, N), 0)
    mask = (gid == bid).astype(_f32)
    o_ref[...] = jnp.dot(mask, z_ref[...], preferred_element_type=_f32,
                         precision=lax.Precision.HIGHEST)


_tc_pool = pl.pallas_call(
    _pool_body, out_shape=jax.ShapeDtypeStruct((G, 3 * H), _f32))


def kernel(x, edge_index, batch, params):
    src = edge_index[0]
    dst = edge_index[1]
    pad = jnp.arange(E_PAD - E, dtype=jnp.int32)
    src_p = jnp.concatenate([src, pad % N]).reshape(NW, SG, SGC, K)
    dst_p = jnp.concatenate([dst, pad % (N_SP - N) + N]).reshape(NW, SG, SGC, K)
    sd_p = jnp.stack([src_p, dst_p], axis=2)
    zeros = jnp.zeros((ROWZ, H), _f32)

    def vec(v):
        return v.reshape(1, H)

    p = _sc_agg(x, sd_p, zeros)
    pm = params[0]
    h, u16 = _tc_layer0(x, p, pm["W1"], vec(pm["b1"]), pm["W2"], vec(pm["b2"]),
                        vec(pm["gamma"]), vec(pm["beta"]))
    hs = [h]
    for l in (1, 2):
        p = _sc_agg(h, sd_p, zeros)
        pm = params[l]
        h = _tc_layer(u16, h, p, pm["W1"][:ID_DIM], pm["W1"][ID_DIM:],
                      vec(pm["b1"]), pm["W2"], vec(pm["b2"]),
                      vec(pm["gamma"]), vec(pm["beta"]))
        hs.append(h)

    z_out = jnp.concatenate(hs, axis=1)
    g_out = _tc_pool(z_out, batch.reshape(1, N))
    return (z_out, g_out)


# speedup vs baseline: 12.6837x; 1.0330x over previous
import functools

import jax
import jax.numpy as jnp
from jax import lax
from jax.experimental import pallas as pl
from jax.experimental.pallas import tpu as pltpu
from jax.experimental.pallas import tpu_sc as plsc

N = 10000
E = 320000
D = 128
H = 128
ID_DIM = 16
G = 64

NC = 2
NS = 16
NW = NC * NS

K = 112
NBUF = 3
SGC = 8
SG = 2 * (-(-E // (NW * K * SGC * 2)))
CH = SG * SGC
PT = CH * K
E_PAD = NW * PT
N_SP = 10112
ROWZ = N_SP // NS

_f32 = jnp.float32


def _sc_mesh():
    return plsc.VectorSubcoreMesh(
        core_axis_name="c", subcore_axis_name="s", num_cores=NC, num_subcores=NS
    )


@functools.partial(
    pl.kernel,
    out_type=jax.ShapeDtypeStruct((NC, N_SP, H), _f32),
    mesh=_sc_mesh(),
    scratch_types=[
        pltpu.VMEM((2, SGC, K), jnp.int32),
        pltpu.VMEM((2, SGC, K), jnp.int32),
        pltpu.VMEM((K, H), _f32),
        pltpu.VMEM((K, H), _f32),
        pltpu.VMEM((K, H), _f32),
        pltpu.VMEM_SHARED((N_SP, H), _f32),
        pltpu.SemaphoreType.DMA,
        pltpu.SemaphoreType.DMA,
        pltpu.SemaphoreType.DMA,
        pltpu.SemaphoreType.DMA,
        pltpu.SemaphoreType.DMA,
    ],
)
def _sc_agg(z_hbm, sd_hbm, zeros_hbm, out_hbm,
            idx0, idx1, rv0, rv1, rv2, agg_sh, s0, s1, s2, i0, i1):
    rows_v = (rv0, rv1, rv2)
    sems = (s0, s1, s2)
    idx = (idx0, idx1)
    isems = (i0, i1)
    cid = lax.axis_index("c")
    sid = lax.axis_index("s")
    wid = cid * NS + sid
    pltpu.sync_copy(zeros_hbm, agg_sh.at[pl.ds(sid * ROWZ, ROWZ)])
    plsc.subcore_barrier()

    pltpu.async_copy(sd_hbm.at[wid, 0], idx0, i0).wait()
    for b in range(NBUF):
        pltpu.async_copy(z_hbm.at[idx0.at[0, b]], rows_v[b], sems[b])
    pltpu.async_copy(sd_hbm.at[wid, 1], idx1, i1)

    def window(i, carry):
        nxt = jnp.minimum(2 * i + 2, SG - 1)
        for half in range(2):
            cur = idx[half]
            oth = idx[1 - half]
            for c in range(SGC):
                b = c % NBUF
                pltpu.make_async_copy(
                    z_hbm.at[pl.ds(0, K)], rows_v[b], sems[b]).wait()
                pltpu.sync_copy(rows_v[b], agg_sh.at[cur.at[1, c]], add=True)
                if c + NBUF < SGC:
                    pltpu.async_copy(
                        z_hbm.at[cur.at[0, c + NBUF]], rows_v[b], sems[b])
                else:
                    cn = c + NBUF - SGC
                    if half == 0:
                        if cn == 0:
                            pltpu.make_async_copy(
                                sd_hbm.at[wid, 0], oth, isems[1]).wait()
                        pltpu.async_copy(
                            z_hbm.at[oth.at[0, cn]], rows_v[b], sems[b])
                    else:

                        @pl.when(i < SG // 2 - 1)
                        def _():
                            if cn == 0:
                                pltpu.make_async_copy(
                                    sd_hbm.at[wid, 0], oth, isems[0]).wait()
                            pltpu.async_copy(
                                z_hbm.at[oth.at[0, cn]], rows_v[b], sems[b])
            if half == 0:
                @pl.when(i < SG // 2 - 1)
                def _():
                    pltpu.async_copy(sd_hbm.at[wid, nxt], idx0, i0)

        @pl.when(i < SG // 2 - 1)
        def _():
            pltpu.async_copy(sd_hbm.at[wid, nxt + 1], idx1, i1)

        return carry

    lax.fori_loop(0, SG // 2, window, 0)
    plsc.subcore_barrier()
    pltpu.sync_copy(agg_sh.at[pl.ds(sid * ROWZ, ROWZ)],
                    out_hbm.at[cid, pl.ds(sid * ROWZ, ROWZ)])


def _mlp_bn(t, w2, b2, gamma, beta):
    h2 = jnp.dot(t, w2, preferred_element_type=_f32) + b2
    mean = jnp.mean(h2, axis=0, keepdims=True)
    cen = h2 - mean
    var = jnp.mean(cen * cen, axis=0, keepdims=True)
    return jnp.maximum(cen * lax.rsqrt(var + 1e-5) * gamma + beta, 0.0)


def _layer0_body(x_ref, p_ref, w1_ref, b1_ref, w2_ref, b2_ref, g_ref, be_ref,
                 h_ref, u16_ref):
    pre = x_ref[...] + p_ref[0, :N] + p_ref[1, :N]
    t = jnp.maximum(jnp.dot(pre, w1_ref[...], preferred_element_type=_f32)
                    + b1_ref[...], 0.0)
    h_ref[...] = _mlp_bn(t, w2_ref[...], b2_ref[...], g_ref[...], be_ref[...])
    u16_ref[...] = pre[:, :ID_DIM]


_tc_layer0 = pl.pallas_call(
    _layer0_body,
    out_shape=(jax.ShapeDtypeStruct((N, H), _f32),
               jax.ShapeDtypeStruct((N, ID_DIM), _f32)))


def _layer_body(u16_ref, h_ref, p_ref, w1a_ref, w1b_ref, b1_ref,
                w2_ref, b2_ref, g_ref, be_ref, o_ref):
    pre = h_ref[...] + p_ref[0, :N] + p_ref[1, :N]
    t = jnp.maximum(
        jnp.dot(u16_ref[...], w1a_ref[...], preferred_element_type=_f32)
        + jnp.dot(pre, w1b_ref[...], preferred_element_type=_f32)
        + b1_ref[...], 0.0)
    o_ref[...] = _mlp_bn(t, w2_ref[...], b2_ref[...], g_ref[...], be_ref[...])


_tc_layer = pl.pallas_call(
    _layer_body, out_shape=jax.ShapeDtypeStruct((N, H), _f32))


def _pool_body(z_ref, b_ref, o_ref):
    bid = b_ref[...]
    gid = lax.broadcasted_iota(jnp.int32, (G, N), 0)
    mask = (gid == bid).astype(_f32)
    o_ref[...] = jnp.dot(mask, z_ref[...], preferred_element_type=_f32,
                         precision=lax.Precision.HIGHEST)


_tc_pool = pl.pallas_call(
    _pool_body, out_shape=jax.ShapeDtypeStruct((G, H), _f32))


def kernel(x, edge_index, batch, params):
    src = edge_index[0]
    dst = edge_index[1]
    pad = jnp.arange(E_PAD - E, dtype=jnp.int32)
    src_p = jnp.concatenate([src, pad % N]).reshape(NW, SG, SGC, K)
    dst_p = jnp.concatenate([dst, pad % (N_SP - N) + N]).reshape(NW, SG, SGC, K)
    sd_p = jnp.stack([src_p, dst_p], axis=2)
    zeros = jnp.zeros((ROWZ, H), _f32)

    def vec(v):
        return v.reshape(1, H)

    p = _sc_agg(x, sd_p, zeros)
    pm = params[0]
    h, u16 = _tc_layer0(x, p, pm["W1"], vec(pm["b1"]), pm["W2"], vec(pm["b2"]),
                        vec(pm["gamma"]), vec(pm["beta"]))
    hs = [h]
    for l in (1, 2):
        p = _sc_agg(h, sd_p, zeros)
        pm = params[l]
        h = _tc_layer(u16, h, p, pm["W1"][:ID_DIM], pm["W1"][ID_DIM:],
                      vec(pm["b1"]), pm["W2"], vec(pm["b2"]),
                      vec(pm["gamma"]), vec(pm["beta"]))
        hs.append(h)

    bmat = batch.reshape(1, N)
    z_out = jnp.concatenate(hs, axis=1)
    g_out = jnp.concatenate([_tc_pool(hh, bmat) for hh in hs], axis=1)
    return (z_out, g_out)


# speedup vs baseline: 13.1562x; 1.0373x over previous
import functools

import jax
import jax.numpy as jnp
from jax import lax
from jax.experimental import pallas as pl
from jax.experimental.pallas import tpu as pltpu
from jax.experimental.pallas import tpu_sc as plsc

N = 10000
E = 320000
D = 128
H = 128
ID_DIM = 16
G = 64

NC = 2
NS = 16
NW = NC * NS

K = 112
NBUF = 3
SGC = 6
SG = 2 * (-(-E // (NW * K * SGC * 2)))
CH = SG * SGC
PT = CH * K
E_PAD = NW * PT
N_SP = 10112
ROWZ = N_SP // NS

_f32 = jnp.float32


def _sc_mesh():
    return plsc.VectorSubcoreMesh(
        core_axis_name="c", subcore_axis_name="s", num_cores=NC, num_subcores=NS
    )


@functools.partial(
    pl.kernel,
    out_type=jax.ShapeDtypeStruct((NC, N_SP, H), _f32),
    mesh=_sc_mesh(),
    scratch_types=[
        pltpu.VMEM((2, SGC, K), jnp.int32),
        pltpu.VMEM((2, SGC, K), jnp.int32),
        pltpu.VMEM((K, H), _f32),
        pltpu.VMEM((K, H), _f32),
        pltpu.VMEM((K, H), _f32),
        pltpu.VMEM_SHARED((N_SP, H), _f32),
        pltpu.SemaphoreType.DMA,
        pltpu.SemaphoreType.DMA,
        pltpu.SemaphoreType.DMA,
        pltpu.SemaphoreType.DMA,
        pltpu.SemaphoreType.DMA,
    ],
)
def _sc_agg(z_hbm, sd_hbm, zeros_hbm, out_hbm,
            idx0, idx1, rv0, rv1, rv2, agg_sh, s0, s1, s2, i0, i1):
    rows_v = (rv0, rv1, rv2)
    sems = (s0, s1, s2)
    idx = (idx0, idx1)
    isems = (i0, i1)
    cid = lax.axis_index("c")
    sid = lax.axis_index("s")
    wid = cid * NS + sid
    pltpu.sync_copy(zeros_hbm, agg_sh.at[pl.ds(sid * ROWZ, ROWZ)])
    plsc.subcore_barrier()

    pltpu.async_copy(sd_hbm.at[wid, 0], idx0, i0).wait()
    for b in range(NBUF):
        pltpu.async_copy(z_hbm.at[idx0.at[0, b]], rows_v[b], sems[b])
    pltpu.async_copy(sd_hbm.at[wid, 1], idx1, i1)

    def window(i, carry):
        nxt = jnp.minimum(2 * i + 2, SG - 1)
        for half in range(2):
            cur = idx[half]
            oth = idx[1 - half]
            for c in range(SGC):
                b = c % NBUF
                pltpu.make_async_copy(
                    z_hbm.at[pl.ds(0, K)], rows_v[b], sems[b]).wait()
                pltpu.sync_copy(rows_v[b], agg_sh.at[cur.at[1, c]], add=True)
                if c + NBUF < SGC:
                    pltpu.async_copy(
                        z_hbm.at[cur.at[0, c + NBUF]], rows_v[b], sems[b])
                else:
                    cn = c + NBUF - SGC
                    if half == 0:
                        if cn == 0:
                            pltpu.make_async_copy(
                                sd_hbm.at[wid, 0], oth, isems[1]).wait()
                        pltpu.async_copy(
                            z_hbm.at[oth.at[0, cn]], rows_v[b], sems[b])
                    else:

                        @pl.when(i < SG // 2 - 1)
                        def _():
                            if cn == 0:
                                pltpu.make_async_copy(
                                    sd_hbm.at[wid, 0], oth, isems[0]).wait()
                            pltpu.async_copy(
                                z_hbm.at[oth.at[0, cn]], rows_v[b], sems[b])
            if half == 0:
                @pl.when(i < SG // 2 - 1)
                def _():
                    pltpu.async_copy(sd_hbm.at[wid, nxt], idx0, i0)

        @pl.when(i < SG // 2 - 1)
        def _():
            pltpu.async_copy(sd_hbm.at[wid, nxt + 1], idx1, i1)

        return carry

    lax.fori_loop(0, SG // 2, window, 0)
    plsc.subcore_barrier()
    pltpu.sync_copy(agg_sh.at[pl.ds(sid * ROWZ, ROWZ)],
                    out_hbm.at[cid, pl.ds(sid * ROWZ, ROWZ)])


def _mlp_bn(t, w2, b2, gamma, beta):
    h2 = jnp.dot(t, w2, preferred_element_type=_f32) + b2
    mean = jnp.mean(h2, axis=0, keepdims=True)
    cen = h2 - mean
    var = jnp.mean(cen * cen, axis=0, keepdims=True)
    return jnp.maximum(cen * lax.rsqrt(var + 1e-5) * gamma + beta, 0.0)


def _layer0_body(x_ref, p_ref, w1_ref, b1_ref, w2_ref, b2_ref, g_ref, be_ref,
                 h_ref, u16_ref):
    pre = x_ref[...] + p_ref[0, :N] + p_ref[1, :N]
    t = jnp.maximum(jnp.dot(pre, w1_ref[...], preferred_element_type=_f32)
                    + b1_ref[...], 0.0)
    h_ref[...] = _mlp_bn(t, w2_ref[...], b2_ref[...], g_ref[...], be_ref[...])
    u16_ref[...] = pre[:, :ID_DIM]


_tc_layer0 = pl.pallas_call(
    _layer0_body,
    out_shape=(jax.ShapeDtypeStruct((N, H), _f32),
               jax.ShapeDtypeStruct((N, ID_DIM), _f32)))


def _layer_body(u16_ref, h_ref, p_ref, w1a_ref, w1b_ref, b1_ref,
                w2_ref, b2_ref, g_ref, be_ref, o_ref):
    pre = h_ref[...] + p_ref[0, :N] + p_ref[1, :N]
    t = jnp.maximum(
        jnp.dot(u16_ref[...], w1a_ref[...], preferred_element_type=_f32)
        + jnp.dot(pre, w1b_ref[...], preferred_element_type=_f32)
        + b1_ref[...], 0.0)
    o_ref[...] = _mlp_bn(t, w2_ref[...], b2_ref[...], g_ref[...], be_ref[...])


_tc_layer = pl.pallas_call(
    _layer_body, out_shape=jax.ShapeDtypeStruct((N, H), _f32))


def _pool_body(z_ref, b_ref, o_ref):
    bid = b_ref[...]
    gid = lax.broadcasted_iota(jnp.int32, (G, N), 0)
    mask = (gid == bid).astype(_f32)
    o_ref[...] = jnp.dot(mask, z_ref[...], preferred_element_type=_f32,
                         precision=lax.Precision.HIGHEST)


_tc_pool = pl.pallas_call(
    _pool_body, out_shape=jax.ShapeDtypeStruct((G, H), _f32))


def kernel(x, edge_index, batch, params):
    src = edge_index[0]
    dst = edge_index[1]
    pad = jnp.arange(E_PAD - E, dtype=jnp.int32)
    src_p = jnp.concatenate([src, pad % N]).reshape(NW, SG, SGC, K)
    dst_p = jnp.concatenate([dst, pad % (N_SP - N) + N]).reshape(NW, SG, SGC, K)
    sd_p = jnp.stack([src_p, dst_p], axis=2)
    zeros = jnp.zeros((ROWZ, H), _f32)

    def vec(v):
        return v.reshape(1, H)

    p = _sc_agg(x, sd_p, zeros)
    pm = params[0]
    h, u16 = _tc_layer0(x, p, pm["W1"], vec(pm["b1"]), pm["W2"], vec(pm["b2"]),
                        vec(pm["gamma"]), vec(pm["beta"]))
    hs = [h]
    for l in (1, 2):
        p = _sc_agg(h, sd_p, zeros)
        pm = params[l]
        h = _tc_layer(u16, h, p, pm["W1"][:ID_DIM], pm["W1"][ID_DIM:],
                      vec(pm["b1"]), pm["W2"], vec(pm["b2"]),
                      vec(pm["gamma"]), vec(pm["beta"]))
        hs.append(h)

    bmat = batch.reshape(1, N)
    z_out = jnp.concatenate(hs, axis=1)
    g_out = jnp.concatenate([_tc_pool(hh, bmat) for hh in hs], axis=1)
    return (z_out, g_out)


# speedup vs baseline: 13.1587x; 1.0002x over previous
import functools

import jax
import jax.numpy as jnp
from jax import lax
from jax.experimental import pallas as pl
from jax.experimental.pallas import tpu as pltpu
from jax.experimental.pallas import tpu_sc as plsc

N = 10000
E = 320000
D = 128
H = 128
ID_DIM = 16
G = 64

NC = 2
NS = 16
NW = NC * NS

K = 112
NBUF = 3
SGC = 6
SG = 2 * (-(-E // (NW * K * SGC * 2)))
CH = SG * SGC
PT = CH * K
E_PAD = NW * PT
N_SP = 10112
ROWZ = N_SP // NS

_f32 = jnp.float32


def _sc_mesh():
    return plsc.VectorSubcoreMesh(
        core_axis_name="c", subcore_axis_name="s", num_cores=NC, num_subcores=NS
    )


@functools.partial(
    pl.kernel,
    out_type=jax.ShapeDtypeStruct((NC, N_SP, H), _f32),
    mesh=_sc_mesh(),
    scratch_types=[
        pltpu.VMEM((2, SGC, K), jnp.int32),
        pltpu.VMEM((2, SGC, K), jnp.int32),
        pltpu.VMEM((K, H), _f32),
        pltpu.VMEM((K, H), _f32),
        pltpu.VMEM((K, H), _f32),
        pltpu.VMEM_SHARED((N_SP, H), _f32),
        pltpu.SemaphoreType.DMA,
        pltpu.SemaphoreType.DMA,
        pltpu.SemaphoreType.DMA,
        pltpu.SemaphoreType.DMA,
        pltpu.SemaphoreType.DMA,
    ],
)
def _sc_agg(z_hbm, sd_hbm, zeros_hbm, out_hbm,
            idx0, idx1, rv0, rv1, rv2, agg_sh, s0, s1, s2, i0, i1):
    rows_v = (rv0, rv1, rv2)
    sems = (s0, s1, s2)
    idx = (idx0, idx1)
    isems = (i0, i1)
    cid = lax.axis_index("c")
    sid = lax.axis_index("s")
    wid = cid * NS + sid
    pltpu.sync_copy(zeros_hbm, agg_sh.at[pl.ds(sid * ROWZ, ROWZ)])
    plsc.subcore_barrier()

    pltpu.async_copy(sd_hbm.at[wid, 0], idx0, i0).wait()
    for b in range(NBUF):
        pltpu.async_copy(z_hbm.at[idx0.at[0, b]], rows_v[b], sems[b])
    pltpu.async_copy(sd_hbm.at[wid, 1], idx1, i1)

    def window(i, carry):
        nxt = jnp.minimum(2 * i + 2, SG - 1)
        for half in range(2):
            cur = idx[half]
            oth = idx[1 - half]
            for c in range(SGC):
                b = c % NBUF
                pltpu.make_async_copy(
                    z_hbm.at[pl.ds(0, K)], rows_v[b], sems[b]).wait()
                pltpu.sync_copy(rows_v[b], agg_sh.at[cur.at[1, c]], add=True)
                if c + NBUF < SGC:
                    pltpu.async_copy(
                        z_hbm.at[cur.at[0, c + NBUF]], rows_v[b], sems[b])
                else:
                    cn = c + NBUF - SGC
                    if half == 0:
                        if cn == 0:
                            pltpu.make_async_copy(
                                sd_hbm.at[wid, 0], oth, isems[1]).wait()
                        pltpu.async_copy(
                            z_hbm.at[oth.at[0, cn]], rows_v[b], sems[b])
                    else:

                        @pl.when(i < SG // 2 - 1)
                        def _():
                            if cn == 0:
                                pltpu.make_async_copy(
                                    sd_hbm.at[wid, 0], oth, isems[0]).wait()
                            pltpu.async_copy(
                                z_hbm.at[oth.at[0, cn]], rows_v[b], sems[b])
            if half == 0:
                @pl.when(i < SG // 2 - 1)
                def _():
                    pltpu.async_copy(sd_hbm.at[wid, nxt], idx0, i0)

        @pl.when(i < SG // 2 - 1)
        def _():
            pltpu.async_copy(sd_hbm.at[wid, nxt + 1], idx1, i1)

        return carry

    lax.fori_loop(0, SG // 2, window, 0)
    plsc.subcore_barrier()
    pltpu.sync_copy(agg_sh.at[pl.ds(sid * ROWZ, ROWZ)],
                    out_hbm.at[cid, pl.ds(sid * ROWZ, ROWZ)])


def _mlp_bn(t, w2, b2, gamma, beta):
    h2 = jnp.dot(t, w2, preferred_element_type=_f32) + b2
    mean = jnp.mean(h2, axis=0, keepdims=True)
    cen = h2 - mean
    var = jnp.mean(cen * cen, axis=0, keepdims=True)
    return jnp.maximum(cen * lax.rsqrt(var + 1e-5) * gamma + beta, 0.0)


def _pool(h, bid):
    gid = lax.broadcasted_iota(jnp.int32, (G, N), 0)
    mask = (gid == bid).astype(_f32)
    return jnp.dot(mask, h, preferred_element_type=_f32,
                   precision=lax.Precision.HIGHEST)


def _layer0_body(x_ref, p_ref, b_ref, w1_ref, b1_ref, w2_ref, b2_ref,
                 g_ref, be_ref, h_ref, u16_ref, g_out_ref):
    pre = x_ref[...] + p_ref[0, :N] + p_ref[1, :N]
    t = jnp.maximum(jnp.dot(pre, w1_ref[...], preferred_element_type=_f32)
                    + b1_ref[...], 0.0)
    h = _mlp_bn(t, w2_ref[...], b2_ref[...], g_ref[...], be_ref[...])
    h_ref[...] = h
    u16_ref[...] = pre[:, :ID_DIM]
    g_out_ref[...] = _pool(h, b_ref[...])


_tc_layer0 = pl.pallas_call(
    _layer0_body,
    out_shape=(jax.ShapeDtypeStruct((N, H), _f32),
               jax.ShapeDtypeStruct((N, ID_DIM), _f32),
               jax.ShapeDtypeStruct((G, H), _f32)))


def _layer_body(u16_ref, h_ref, p_ref, b_ref, w1a_ref, w1b_ref, b1_ref,
                w2_ref, b2_ref, g_ref, be_ref, o_ref, g_out_ref):
    pre = h_ref[...] + p_ref[0, :N] + p_ref[1, :N]
    t = jnp.maximum(
        jnp.dot(u16_ref[...], w1a_ref[...], preferred_element_type=_f32)
        + jnp.dot(pre, w1b_ref[...], preferred_element_type=_f32)
        + b1_ref[...], 0.0)
    h = _mlp_bn(t, w2_ref[...], b2_ref[...], g_ref[...], be_ref[...])
    o_ref[...] = h
    g_out_ref[...] = _pool(h, b_ref[...])


_tc_layer = pl.pallas_call(
    _layer_body, out_shape=(jax.ShapeDtypeStruct((N, H), _f32),
                            jax.ShapeDtypeStruct((G, H), _f32)))


def kernel(x, edge_index, batch, params):
    src = edge_index[0]
    dst = edge_index[1]
    pad = jnp.arange(E_PAD - E, dtype=jnp.int32)
    src_p = jnp.concatenate([src, pad % N]).reshape(NW, SG, SGC, K)
    dst_p = jnp.concatenate([dst, pad % (N_SP - N) + N]).reshape(NW, SG, SGC, K)
    sd_p = jnp.stack([src_p, dst_p], axis=2)
    zeros = jnp.zeros((ROWZ, H), _f32)

    def vec(v):
        return v.reshape(1, H)

    bmat = batch.reshape(1, N)
    p = _sc_agg(x, sd_p, zeros)
    pm = params[0]
    h, u16, g = _tc_layer0(x, p, bmat, pm["W1"], vec(pm["b1"]), pm["W2"],
                           vec(pm["b2"]), vec(pm["gamma"]), vec(pm["beta"]))
    hs, gs = [h], [g]
    for l in (1, 2):
        p = _sc_agg(h, sd_p, zeros)
        pm = params[l]
        h, g = _tc_layer(u16, h, p, bmat, pm["W1"][:ID_DIM], pm["W1"][ID_DIM:],
                         vec(pm["b1"]), pm["W2"], vec(pm["b2"]),
                         vec(pm["gamma"]), vec(pm["beta"]))
        hs.append(h)
        gs.append(g)

    z_out = jnp.concatenate(hs, axis=1)
    g_out = jnp.concatenate(gs, axis=1)
    return (z_out, g_out)


# speedup vs baseline: 13.5879x; 1.0326x over previous
import functools

import jax
import jax.numpy as jnp
from jax import lax
from jax.experimental import pallas as pl
from jax.experimental.pallas import tpu as pltpu
from jax.experimental.pallas import tpu_sc as plsc

N = 10000
E = 320000
D = 128
H = 128
ID_DIM = 16
G = 64

NC = 2
NS = 16
NW = NC * NS

K = 80
NBUF = 4
SGC = 8
SG = 2 * (-(-E // (NW * K * SGC * 2)))
CH = SG * SGC
PT = CH * K
E_PAD = NW * PT
N_SP = 10112
ROWZ = N_SP // NS

_f32 = jnp.float32


def _sc_mesh():
    return plsc.VectorSubcoreMesh(
        core_axis_name="c", subcore_axis_name="s", num_cores=NC, num_subcores=NS
    )


@functools.partial(
    pl.kernel,
    out_type=jax.ShapeDtypeStruct((NC, N_SP, H), _f32),
    mesh=_sc_mesh(),
    scratch_types=[
        pltpu.VMEM((2, SGC, K), jnp.int32),
        pltpu.VMEM((2, SGC, K), jnp.int32),
        pltpu.VMEM((K, H), _f32),
        pltpu.VMEM((K, H), _f32),
        pltpu.VMEM((K, H), _f32),
        pltpu.VMEM((K, H), _f32),
        pltpu.VMEM_SHARED((N_SP, H), _f32),
        pltpu.SemaphoreType.DMA,
        pltpu.SemaphoreType.DMA,
        pltpu.SemaphoreType.DMA,
        pltpu.SemaphoreType.DMA,
        pltpu.SemaphoreType.DMA,
        pltpu.SemaphoreType.DMA,
    ],
)
def _sc_agg(z_hbm, sd_hbm, zeros_hbm, out_hbm,
            idx0, idx1, rv0, rv1, rv2, rv3, agg_sh, s0, s1, s2, s3, i0, i1):
    rows_v = (rv0, rv1, rv2, rv3)
    sems = (s0, s1, s2, s3)
    idx = (idx0, idx1)
    isems = (i0, i1)
    cid = lax.axis_index("c")
    sid = lax.axis_index("s")
    wid = cid * NS + sid
    pltpu.sync_copy(zeros_hbm, agg_sh.at[pl.ds(sid * ROWZ, ROWZ)])
    plsc.subcore_barrier()

    pltpu.async_copy(sd_hbm.at[wid, 0], idx0, i0).wait()
    for b in range(NBUF):
        pltpu.async_copy(z_hbm.at[idx0.at[0, b]], rows_v[b], sems[b])
    pltpu.async_copy(sd_hbm.at[wid, 1], idx1, i1)

    def window(i, carry):
        nxt = jnp.minimum(2 * i + 2, SG - 1)
        for half in range(2):
            cur = idx[half]
            oth = idx[1 - half]
            for c in range(SGC):
                b = c % NBUF
                pltpu.make_async_copy(
                    z_hbm.at[pl.ds(0, K)], rows_v[b], sems[b]).wait()
                pltpu.sync_copy(rows_v[b], agg_sh.at[cur.at[1, c]], add=True)
                if c + NBUF < SGC:
                    pltpu.async_copy(
                        z_hbm.at[cur.at[0, c + NBUF]], rows_v[b], sems[b])
                else:
                    cn = c + NBUF - SGC
                    if half == 0:
                        if cn == 0:
                            pltpu.make_async_copy(
                                sd_hbm.at[wid, 0], oth, isems[1]).wait()
                        pltpu.async_copy(
                            z_hbm.at[oth.at[0, cn]], rows_v[b], sems[b])
                    else:

                        @pl.when(i < SG // 2 - 1)
                        def _():
                            if cn == 0:
                                pltpu.make_async_copy(
                                    sd_hbm.at[wid, 0], oth, isems[0]).wait()
                            pltpu.async_copy(
                                z_hbm.at[oth.at[0, cn]], rows_v[b], sems[b])
            if half == 0:
                @pl.when(i < SG // 2 - 1)
                def _():
                    pltpu.async_copy(sd_hbm.at[wid, nxt], idx0, i0)

        @pl.when(i < SG // 2 - 1)
        def _():
            pltpu.async_copy(sd_hbm.at[wid, nxt + 1], idx1, i1)

        return carry

    lax.fori_loop(0, SG // 2, window, 0)
    plsc.subcore_barrier()
    pltpu.sync_copy(agg_sh.at[pl.ds(sid * ROWZ, ROWZ)],
                    out_hbm.at[cid, pl.ds(sid * ROWZ, ROWZ)])


def _mlp_bn(t, w2, b2, gamma, beta):
    h2 = jnp.dot(t, w2, preferred_element_type=_f32) + b2
    mean = jnp.mean(h2, axis=0, keepdims=True)
    cen = h2 - mean
    var = jnp.mean(cen * cen, axis=0, keepdims=True)
    return jnp.maximum(cen * lax.rsqrt(var + 1e-5) * gamma + beta, 0.0)


def _pool(h, bid):
    gid = lax.broadcasted_iota(jnp.int32, (G, N), 0)
    mask = (gid == bid).astype(_f32)
    return jnp.dot(mask, h, preferred_element_type=_f32,
                   precision=lax.Precision.HIGHEST)


def _layer0_body(x_ref, p_ref, b_ref, w1_ref, b1_ref, w2_ref, b2_ref,
                 g_ref, be_ref, h_ref, u16_ref, g_out_ref):
    pre = x_ref[...] + p_ref[0, :N] + p_ref[1, :N]
    t = jnp.maximum(jnp.dot(pre, w1_ref[...], preferred_element_type=_f32)
                    + b1_ref[...], 0.0)
    h = _mlp_bn(t, w2_ref[...], b2_ref[...], g_ref[...], be_ref[...])
    h_ref[...] = h
    u16_ref[...] = pre[:, :ID_DIM]
    g_out_ref[...] = _pool(h, b_ref[...])


_tc_layer0 = pl.pallas_call(
    _layer0_body,
    out_shape=(jax.ShapeDtypeStruct((N, H), _f32),
               jax.ShapeDtypeStruct((N, ID_DIM), _f32),
               jax.ShapeDtypeStruct((G, H), _f32)))


def _layer_body(u16_ref, h_ref, p_ref, b_ref, w1a_ref, w1b_ref, b1_ref,
                w2_ref, b2_ref, g_ref, be_ref, o_ref, g_out_ref):
    pre = h_ref[...] + p_ref[0, :N] + p_ref[1, :N]
    t = jnp.maximum(
        jnp.dot(u16_ref[...], w1a_ref[...], preferred_element_type=_f32)
        + jnp.dot(pre, w1b_ref[...], preferred_element_type=_f32)
        + b1_ref[...], 0.0)
    h = _mlp_bn(t, w2_ref[...], b2_ref[...], g_ref[...], be_ref[...])
    o_ref[...] = h
    g_out_ref[...] = _pool(h, b_ref[...])


_tc_layer = pl.pallas_call(
    _layer_body, out_shape=(jax.ShapeDtypeStruct((N, H), _f32),
                            jax.ShapeDtypeStruct((G, H), _f32)))


def kernel(x, edge_index, batch, params):
    src = edge_index[0]
    dst = edge_index[1]
    pad = jnp.arange(E_PAD - E, dtype=jnp.int32)
    src_p = jnp.concatenate([src, pad % N]).reshape(NW, SG, SGC, K)
    dst_p = jnp.concatenate([dst, pad % (N_SP - N) + N]).reshape(NW, SG, SGC, K)
    sd_p = jnp.stack([src_p, dst_p], axis=2)
    zeros = jnp.zeros((ROWZ, H), _f32)

    def vec(v):
        return v.reshape(1, H)

    bmat = batch.reshape(1, N)
    p = _sc_agg(x, sd_p, zeros)
    pm = params[0]
    h, u16, g = _tc_layer0(x, p, bmat, pm["W1"], vec(pm["b1"]), pm["W2"],
                           vec(pm["b2"]), vec(pm["gamma"]), vec(pm["beta"]))
    hs, gs = [h], [g]
    for l in (1, 2):
        p = _sc_agg(h, sd_p, zeros)
        pm = params[l]
        h, g = _tc_layer(u16, h, p, bmat, pm["W1"][:ID_DIM], pm["W1"][ID_DIM:],
                         vec(pm["b1"]), pm["W2"], vec(pm["b2"]),
                         vec(pm["gamma"]), vec(pm["beta"]))
        hs.append(h)
        gs.append(g)

    z_out = jnp.concatenate(hs, axis=1)
    g_out = jnp.concatenate(gs, axis=1)
    return (z_out, g_out)
